# baseline (speedup 1.0000x reference)
import functools

import jax
import jax.numpy as jnp
import numpy as np
from jax import lax
from jax.experimental import pallas as pl
from jax.experimental.pallas import tpu as pltpu
from jax.experimental.pallas import tpu_sc as plsc

V = 100000
D = 128
C = 2
B = 4096
L = 200

NC = 2
NS = 16
NW = NC * NS
BPW = B // NW
HALF = L // 2
NLANE = 16
DW = D // 2
NGRP = DW // NLANE

VPW = V // NW
PCHUNK = 125
NCHUNKS = VPW // PCHUNK

_mesh = plsc.VectorSubcoreMesh(core_axis_name="c", subcore_axis_name="s")
_params = pltpu.CompilerParams(needs_layout_passes=False, use_tc_tiling_on_sc=False)


@functools.partial(
    pl.kernel,
    mesh=_mesh,
    out_type=jax.ShapeDtypeStruct((V, DW), jnp.int32),
    scratch_types=[
        pltpu.VMEM((PCHUNK, D), jnp.float32),
        pltpu.VMEM((PCHUNK, D), jnp.float32),
        pltpu.VMEM((PCHUNK, D), jnp.float32),
        pltpu.VMEM((PCHUNK, D), jnp.float32),
        pltpu.VMEM((PCHUNK, DW), jnp.int32),
        pltpu.VMEM((PCHUNK, DW), jnp.int32),
        pltpu.VMEM((PCHUNK, DW), jnp.int32),
        pltpu.VMEM((PCHUNK, DW), jnp.int32),
        pltpu.SemaphoreType.DMA,
        pltpu.SemaphoreType.DMA,
        pltpu.SemaphoreType.DMA,
        pltpu.SemaphoreType.DMA,
        pltpu.SemaphoreType.DMA,
        pltpu.SemaphoreType.DMA,
        pltpu.SemaphoreType.DMA,
        pltpu.SemaphoreType.DMA,
    ],
    compiler_params=_params,
)
def _pack_sc(emb_hbm, tab_hbm, in0, in1, in2, in3, po0, po1, po2, po3,
             si0, si1, si2, si3, so0, so1, so2, so3):
    wid = lax.axis_index("s") * NC + lax.axis_index("c")
    base = wid * VPW
    ins = (in0, in1, in2, in3)
    pos = (po0, po1, po2, po3)
    sis = (si0, si1, si2, si3)
    sos = (so0, so1, so2, so3)

    def fetch(slot, c):
        pltpu.async_copy(emb_hbm.at[pl.ds(base + c * PCHUNK, PCHUNK)], ins[slot], sis[slot])

    def wait_fetch(slot):
        pltpu.make_async_copy(emb_hbm.at[pl.ds(0, PCHUNK)], ins[slot], sis[slot]).wait()

    def put(slot, c):
        pltpu.async_copy(pos[slot], tab_hbm.at[pl.ds(base + c * PCHUNK, PCHUNK)], sos[slot])

    def wait_put(slot):
        pltpu.make_async_copy(pos[slot], tab_hbm.at[pl.ds(0, PCHUNK)], sos[slot]).wait()

    half = jnp.full((NLANE,), 0x8000, dtype=jnp.int32)
    lomask = jnp.full((NLANE,), 0xFFFF, dtype=jnp.int32)
    himask = jnp.full((NLANE,), np.int32(np.uint32(0xFFFF0000).view(np.int32)),
                      dtype=jnp.int32)

    def pack_chunk(ki, ko):
        src = ins[ki]
        dst = pos[ko]

        def prow(r, carry):
            for g in range(NGRP):
                a = src[r, pl.ds(g * NLANE, NLANE)]
                b = src[r, pl.ds(D // 2 + g * NLANE, NLANE)]
                ai = plsc.bitcast(a, jnp.int32)
                bi = plsc.bitcast(b, jnp.int32)
                lo = ((ai + half) >> 16) & lomask
                hi = (bi + half) & himask
                dst[r, pl.ds(g * NLANE, NLANE)] = lo | hi
            return carry

        lax.fori_loop(0, PCHUNK, prow, 0, unroll=8)

    for p in range(4):
        fetch(p, p)

    @pl.loop(0, NCHUNKS - 1, step=4)
    def _chunk_quad(c0):
        for k in range(4):
            c = c0 + k
            wait_fetch(k)

            @pl.when(c >= 4)
            def _drain():
                wait_put(k)

            pack_chunk(k, k)

            @pl.when(c + 4 < NCHUNKS)
            def _prefetch():
                fetch(k, c + 4)

            put(k, c)

    wait_fetch(0)
    wait_put(0)
    pack_chunk(0, 0)
    put(0, NCHUNKS - 1)
    wait_put(0)
    for p in range(1, 4):
        wait_put(p)


@functools.partial(
    pl.kernel,
    mesh=_mesh,
    out_type=jax.ShapeDtypeStruct((B, D), jnp.float32),
    scratch_types=[
        pltpu.VMEM((2 * BPW, HALF), jnp.int32),
        pltpu.VMEM((L, DW), jnp.int32),
        pltpu.VMEM((L, DW), jnp.int32),
        pltpu.VMEM((L, DW), jnp.int32),
        pltpu.VMEM((L, DW), jnp.int32),
        pltpu.VMEM((BPW, D), jnp.float32),
        pltpu.SemaphoreType.DMA,
        pltpu.SemaphoreType.DMA,
        pltpu.SemaphoreType.DMA,
        pltpu.SemaphoreType.DMA,
    ],
    compiler_params=_params,
)
def _pool_sc(x_hbm, tab_hbm, out_hbm, idx_v, rows0, rows1, rows2, rows3,
             out_v, sem0, sem1, sem2, sem3):
    wid = lax.axis_index("s") * NC + lax.axis_index("c")
    base = wid * BPW
    pltpu.sync_copy(x_hbm.at[pl.ds(2 * base, 2 * BPW)], idx_v)

    rows = (rows0, rows1, rows2, rows3)
    sems = (sem0, sem1, sem2, sem3)

    def issue(slot, row):
        pltpu.async_copy(
            tab_hbm.at[idx_v.at[2 * row]], rows[slot].at[pl.ds(0, HALF)], sems[slot])
        pltpu.async_copy(
            tab_hbm.at[idx_v.at[2 * row + 1]], rows[slot].at[pl.ds(HALF, HALF)], sems[slot])

    def wait(slot):
        pltpu.make_async_copy(tab_hbm.at[pl.ds(0, L)], rows[slot], sems[slot]).wait()

    himask = jnp.full((NLANE,), np.int32(np.uint32(0xFFFF0000).view(np.int32)),
                      dtype=jnp.int32)
    for p in range(3):
        issue(p, p)

    @pl.loop(0, BPW, step=4)
    def _row_quad(b0):
        for k in range(4):
            row = b0 + k
            nxt = row + 3

            @pl.when(nxt < BPW)
            def _prefetch():
                issue((k + 3) % 4, nxt)

            wait(k)
            buf = rows[k]

            def racc(r, accs):
                new = list(accs)
                for g in range(NGRP):
                    w = buf[r, pl.ds(g * NLANE, NLANE)]
                    lo = plsc.bitcast(w << 16, jnp.float32)
                    hi = plsc.bitcast(w & himask, jnp.float32)
                    new[g] = new[g] + lo
                    new[NGRP + g] = new[NGRP + g] + hi
                return tuple(new)

            zeros = tuple(jnp.zeros((NLANE,), jnp.float32) for _ in range(2 * NGRP))
            accs = lax.fori_loop(0, L, racc, zeros, unroll=4)
            for j in range(2 * NGRP):
                out_v[row, pl.ds(j * NLANE, NLANE)] = accs[j]

    pltpu.sync_copy(out_v, out_hbm.at[pl.ds(base, BPW)])


def _mlp_body(h_ref, w1_ref, b1_ref, w2_ref, b2_ref, o_ref):
    h = h_ref[...] * (1.0 / L)
    z = jnp.dot(h, w1_ref[...], preferred_element_type=jnp.float32)
    z = jnp.maximum(z + b1_ref[...], 0.0)
    o_ref[...] = jnp.dot(z, w2_ref[...], preferred_element_type=jnp.float32) + b2_ref[...]


@jax.jit
def kernel(x, emb, W1, b1, W2, b2):
    x2 = x.reshape(2 * B, HALF)
    tab16 = _pack_sc(emb)
    pooled = _pool_sc(x2, tab16)
    out = pl.pallas_call(
        _mlp_body,
        out_shape=jax.ShapeDtypeStruct((B, C), jnp.float32),
    )(pooled, W1, b1.reshape(1, 64), W2, b2.reshape(1, C))
    return out


# speedup vs baseline: 1.0736x; 1.0736x over previous
import functools

import jax
import jax.numpy as jnp
import numpy as np
from jax import lax
from jax.experimental import pallas as pl
from jax.experimental.pallas import tpu as pltpu
from jax.experimental.pallas import tpu_sc as plsc

V = 100000
D = 128
C = 2
B = 4096
L = 200

NC = 2
NS = 16
NW = NC * NS
BPW = B // NW
HALF = L // 2
NLANE = 16
NGRP = D // (2 * NLANE)

VPW = V // NW
PCHUNK = 125
NCHUNKS = VPW // PCHUNK

_mesh = plsc.VectorSubcoreMesh(core_axis_name="c", subcore_axis_name="s")
_params = pltpu.CompilerParams(needs_layout_passes=False, use_tc_tiling_on_sc=False)


@functools.partial(
    pl.kernel,
    mesh=_mesh,
    out_type=jax.ShapeDtypeStruct((V, D), jnp.bfloat16),
    scratch_types=[
        pltpu.VMEM((PCHUNK, D), jnp.float32),
        pltpu.VMEM((PCHUNK, D), jnp.float32),
        pltpu.VMEM((PCHUNK, D), jnp.float32),
        pltpu.VMEM((PCHUNK, D), jnp.float32),
        pltpu.VMEM((PCHUNK, D), jnp.bfloat16),
        pltpu.VMEM((PCHUNK, D), jnp.bfloat16),
        pltpu.VMEM((PCHUNK, D), jnp.bfloat16),
        pltpu.VMEM((PCHUNK, D), jnp.bfloat16),
        pltpu.SemaphoreType.DMA,
        pltpu.SemaphoreType.DMA,
        pltpu.SemaphoreType.DMA,
        pltpu.SemaphoreType.DMA,
        pltpu.SemaphoreType.DMA,
        pltpu.SemaphoreType.DMA,
        pltpu.SemaphoreType.DMA,
        pltpu.SemaphoreType.DMA,
    ],
    compiler_params=_params,
)
def _pack_sc(emb_hbm, tab_hbm, in0, in1, in2, in3, po0, po1, po2, po3,
             si0, si1, si2, si3, so0, so1, so2, so3):
    wid = lax.axis_index("s") * NC + lax.axis_index("c")
    base = wid * VPW
    ins = (in0, in1, in2, in3)
    pos = (po0, po1, po2, po3)
    sis = (si0, si1, si2, si3)
    sos = (so0, so1, so2, so3)

    def fetch(slot, c):
        pltpu.async_copy(emb_hbm.at[pl.ds(base + c * PCHUNK, PCHUNK)], ins[slot], sis[slot])

    def wait_fetch(slot):
        pltpu.make_async_copy(emb_hbm.at[pl.ds(0, PCHUNK)], ins[slot], sis[slot]).wait()

    def put(slot, c):
        pltpu.async_copy(pos[slot], tab_hbm.at[pl.ds(base + c * PCHUNK, PCHUNK)], sos[slot])

    def wait_put(slot):
        pltpu.make_async_copy(pos[slot], tab_hbm.at[pl.ds(0, PCHUNK)], sos[slot]).wait()

    def pack_chunk(ki, ko):
        src = ins[ki]
        dst = pos[ko]

        def prow(r, carry):
            for g in range(NGRP):
                a = src[r, pl.ds(g * NLANE, NLANE)]
                b = src[r, pl.ds(D // 2 + g * NLANE, NLANE)]
                dst[r, pl.ds(g * 2 * NLANE, 2 * NLANE)] = plsc.pack(
                    a, b, format=plsc.PackFormat.INTERLEAVED)
            return carry

        lax.fori_loop(0, PCHUNK, prow, 0, unroll=8)

    for p in range(4):
        fetch(p, p)

    @pl.loop(0, NCHUNKS - 1, step=4)
    def _chunk_quad(c0):
        for k in range(4):
            c = c0 + k
            wait_fetch(k)

            @pl.when(c >= 4)
            def _drain():
                wait_put(k)

            pack_chunk(k, k)

            @pl.when(c + 4 < NCHUNKS)
            def _prefetch():
                fetch(k, c + 4)

            put(k, c)

    wait_fetch(0)
    wait_put(0)
    pack_chunk(0, 0)
    put(0, NCHUNKS - 1)
    wait_put(0)
    for p in range(1, 4):
        wait_put(p)


@functools.partial(
    pl.kernel,
    mesh=_mesh,
    out_type=jax.ShapeDtypeStruct((B, D), jnp.float32),
    scratch_types=[
        pltpu.VMEM((2 * BPW, HALF), jnp.int32),
        pltpu.VMEM((L, D), jnp.bfloat16),
        pltpu.VMEM((L, D), jnp.bfloat16),
        pltpu.VMEM((L, D), jnp.bfloat16),
        pltpu.VMEM((L, D), jnp.bfloat16),
        pltpu.VMEM((BPW, D), jnp.float32),
        pltpu.SemaphoreType.DMA,
        pltpu.SemaphoreType.DMA,
        pltpu.SemaphoreType.DMA,
        pltpu.SemaphoreType.DMA,
    ],
    compiler_params=_params,
)
def _pool_sc(x_hbm, tab_hbm, out_hbm, idx_v, rows0, rows1, rows2, rows3,
             out_v, sem0, sem1, sem2, sem3):
    wid = lax.axis_index("s") * NC + lax.axis_index("c")
    base = wid * BPW
    pltpu.sync_copy(x_hbm.at[pl.ds(2 * base, 2 * BPW)], idx_v)

    rows = (rows0, rows1, rows2, rows3)
    sems = (sem0, sem1, sem2, sem3)

    def issue(slot, row):
        pltpu.async_copy(
            tab_hbm.at[idx_v.at[2 * row]], rows[slot].at[pl.ds(0, HALF)], sems[slot])
        pltpu.async_copy(
            tab_hbm.at[idx_v.at[2 * row + 1]], rows[slot].at[pl.ds(HALF, HALF)], sems[slot])

    def wait(slot):
        pltpu.make_async_copy(tab_hbm.at[pl.ds(0, L)], rows[slot], sems[slot]).wait()

    for p in range(3):
        issue(p, p)

    @pl.loop(0, BPW, step=4)
    def _row_quad(b0):
        for k in range(4):
            row = b0 + k
            nxt = row + 3

            @pl.when(nxt < BPW)
            def _prefetch():
                issue((k + 3) % 4, nxt)

            wait(k)
            buf = rows[k]

            def racc(r, accs):
                new = list(accs)
                for g in range(NGRP):
                    wb = buf[r, pl.ds(g * 2 * NLANE, 2 * NLANE)]
                    lo, hi = plsc.unpack(wb, format=plsc.PackFormat.INTERLEAVED)
                    new[g] = new[g] + lo
                    new[NGRP + g] = new[NGRP + g] + hi
                return tuple(new)

            zeros = tuple(jnp.zeros((NLANE,), jnp.float32) for _ in range(2 * NGRP))
            accs = lax.fori_loop(0, L, racc, zeros, unroll=4)
            for j in range(2 * NGRP):
                out_v[row, pl.ds(j * NLANE, NLANE)] = accs[j]

    pltpu.sync_copy(out_v, out_hbm.at[pl.ds(base, BPW)])


def _mlp_body(h_ref, w1_ref, b1_ref, w2_ref, b2_ref, o_ref):
    h = h_ref[...] * (1.0 / L)
    z = jnp.dot(h, w1_ref[...], preferred_element_type=jnp.float32)
    z = jnp.maximum(z + b1_ref[...], 0.0)
    o_ref[...] = jnp.dot(z, w2_ref[...], preferred_element_type=jnp.float32) + b2_ref[...]


@jax.jit
def kernel(x, emb, W1, b1, W2, b2):
    x2 = x.reshape(2 * B, HALF)
    tab16 = _pack_sc(emb)
    pooled = _pool_sc(x2, tab16)
    out = pl.pallas_call(
        _mlp_body,
        out_shape=jax.ShapeDtypeStruct((B, C), jnp.float32),
    )(pooled, W1, b1.reshape(1, 64), W2, b2.reshape(1, C))
    return out
